# baseline (speedup 1.0000x reference)
import jax
import jax.numpy as jnp
from jax import lax
from jax.experimental import pallas as pl
from jax.experimental.pallas import tpu as pltpu
from jax.experimental.pallas import tpu_sc as plsc

NC, NS, L = 2, 16, 16
NW = NC * NS
B, S, D = 4, 2048, 1024
S_PER_W = S // NW
CHUNK = 8
NSC = S_PER_W // CHUNK
NGRP = 3
GROWS = B * CHUNK
LPR = D // L


def _gather_idx(idx_v, g):
    return idx_v.at[g]


def _body(x_hbm, tok_hbm, pos_hbm, out_hbm, idx_v, pos_bufs, tok_bufs,
          in_sems, out_sems, pos_sems, idx_sems):
    wid = lax.axis_index("s") * NC + lax.axis_index("c")
    s0 = pl.multiple_of(wid * S_PER_W, S_PER_W)

    def issue_pos(g):
        src = pos_hbm.at[pl.ds(s0 + g * CHUNK, CHUNK)]
        return pltpu.async_copy(src, pos_bufs.at[g % 2], pos_sems.at[g % 2])

    def issue_gather(g):
        return pltpu.async_copy(tok_hbm.at[_gather_idx(idx_v, g)],
                                tok_bufs.at[g % NGRP], in_sems.at[g % NGRP])

    idx_handles = {}
    for g in range(NSC):
        for b in range(B):
            src = x_hbm.at[b, pl.ds(s0 + g * CHUNK, CHUNK)]
            dst = idx_v.at[g, pl.ds(b * CHUNK, CHUNK)]
            idx_handles[(g, b)] = pltpu.async_copy(src, dst, idx_sems.at[g])
    pos_handles = {0: issue_pos(0), 1: issue_pos(1)}

    gather_handles = {}
    for g in range(2):
        for b in range(B):
            idx_handles.pop((g, b)).wait()
        gather_handles[g] = issue_gather(g)
    out_handles = {}

    for g in range(NSC):
        pos_handles.pop(g).wait()
        gather_handles.pop(g).wait()
        pos = pos_bufs.at[g % 2]
        tok = tok_bufs.at[g % NGRP]

        @plsc.parallel_loop(0, CHUNK * LPR, unroll=4)
        def _(j):
            r = j // LPR
            col = (j - r * LPR) * L
            p = pos[r, pl.ds(col, L)]
            for b in range(B):
                tok[b * CHUNK + r, pl.ds(col, L)] = (
                    tok[b * CHUNK + r, pl.ds(col, L)] + p)

        if g + 2 < NSC:
            pos_handles[g + 2] = issue_pos(g + 2)
        for b in range(B):
            src = tok_bufs.at[g % NGRP, pl.ds(b * CHUNK, CHUNK)]
            dst = out_hbm.at[b, pl.ds(s0 + g * CHUNK, CHUNK)]
            out_handles[(g, b)] = pltpu.async_copy(
                src, dst, out_sems.at[(g % NGRP) * B + b])
        if g + 2 < NSC:
            for b in range(B):
                h = out_handles.pop((g - 1, b), None)
                if h is not None:
                    h.wait()
                idx_handles.pop((g + 2, b)).wait()
            gather_handles[g + 2] = issue_gather(g + 2)

    for h in out_handles.values():
        h.wait()


@jax.jit
def kernel(x, token_table, position_table):
    mesh = plsc.VectorSubcoreMesh(core_axis_name="c", subcore_axis_name="s",
                                  num_cores=NC, num_subcores=NS)
    run = pl.kernel(
        _body,
        out_type=jax.ShapeDtypeStruct((B, S, D), jnp.float32),
        mesh=mesh,
        scratch_types=dict(
            idx_v=pltpu.VMEM((NSC, GROWS), jnp.int32),
            pos_bufs=pltpu.VMEM((2, CHUNK, D), jnp.float32),
            tok_bufs=pltpu.VMEM((NGRP, GROWS, D), jnp.float32),
            in_sems=pltpu.SemaphoreType.DMA((NGRP,)),
            out_sems=pltpu.SemaphoreType.DMA((NGRP * B,)),
            pos_sems=pltpu.SemaphoreType.DMA((2,)),
            idx_sems=pltpu.SemaphoreType.DMA((NSC,)),
        ),
    )
    return run(x.astype(jnp.int32), token_table, position_table)


# speedup vs baseline: 1.0389x; 1.0389x over previous
import jax
import jax.numpy as jnp
from jax import lax
from jax.experimental import pallas as pl
from jax.experimental.pallas import tpu as pltpu
from jax.experimental.pallas import tpu_sc as plsc

NC, NS, L = 2, 16, 16
NW = NC * NS
B, S, D = 4, 2048, 1024
S_PER_W = S // NW
CHUNK = 8
NSC = S_PER_W // CHUNK
NGRP = 3
LPR = D // L


def _idx_slice(idx_v, b, sc):
    return idx_v.at[b, pl.ds(sc * CHUNK, CHUNK)]


def _body(x_hbm, tok_hbm, pos_hbm, out_hbm, idx_v, pos_bufs, tok_bufs,
          in_sems, out_sems, pos_sems, idx_sems):
    wid = lax.axis_index("s") * NC + lax.axis_index("c")
    s0 = pl.multiple_of(wid * S_PER_W, S_PER_W)

    def issue_pos(g):
        src = pos_hbm.at[pl.ds(s0 + g * CHUNK, CHUNK)]
        return pltpu.async_copy(src, pos_bufs.at[g % 2], pos_sems.at[g % 2])

    def issue_gather(g, b):
        slot = (g % NGRP) * B + b
        return pltpu.async_copy(tok_hbm.at[_idx_slice(idx_v, b, g)],
                                tok_bufs.at[slot], in_sems.at[slot])

    idx_handles = [
        pltpu.async_copy(x_hbm.at[b, pl.ds(s0, S_PER_W)], idx_v.at[b],
                         idx_sems.at[b])
        for b in range(B)
    ]
    pos_handles = {0: issue_pos(0), 1: issue_pos(1)}
    for h in idx_handles:
        h.wait()

    gather_handles = {}
    for g in range(2):
        for b in range(B):
            gather_handles[(g, b)] = issue_gather(g, b)
    out_handles = {}

    for g in range(NSC):
        pos_handles.pop(g).wait()
        for b in range(B):
            gather_handles.pop((g, b)).wait()
        pos = pos_bufs.at[g % 2]
        toks = [tok_bufs.at[(g % NGRP) * B + b] for b in range(B)]

        @plsc.parallel_loop(0, CHUNK * LPR, unroll=4)
        def _(j):
            r = j // LPR
            col = (j - r * LPR) * L
            p = pos[r, pl.ds(col, L)]
            for t in toks:
                t[r, pl.ds(col, L)] = t[r, pl.ds(col, L)] + p

        if g + 2 < NSC:
            pos_handles[g + 2] = issue_pos(g + 2)
        for b in range(B):
            slot = (g % NGRP) * B + b
            dst = out_hbm.at[b, pl.ds(s0 + g * CHUNK, CHUNK)]
            out_handles[(g, b)] = pltpu.async_copy(tok_bufs.at[slot], dst,
                                                   out_sems.at[slot])
        if g + 2 < NSC:
            for b in range(B):
                h = out_handles.pop((g - 1, b), None)
                if h is not None:
                    h.wait()
                gather_handles[(g + 2, b)] = issue_gather(g + 2, b)

    for h in out_handles.values():
        h.wait()


@jax.jit
def kernel(x, token_table, position_table):
    mesh = plsc.VectorSubcoreMesh(core_axis_name="c", subcore_axis_name="s",
                                  num_cores=NC, num_subcores=NS)
    run = pl.kernel(
        _body,
        out_type=jax.ShapeDtypeStruct((B, S, D), jnp.float32),
        mesh=mesh,
        scratch_types=dict(
            idx_v=pltpu.VMEM((B, S_PER_W), jnp.int32),
            pos_bufs=pltpu.VMEM((2, CHUNK, D), jnp.float32),
            tok_bufs=pltpu.VMEM((NGRP * B, CHUNK, D), jnp.float32),
            in_sems=pltpu.SemaphoreType.DMA((NGRP * B,)),
            out_sems=pltpu.SemaphoreType.DMA((NGRP * B,)),
            pos_sems=pltpu.SemaphoreType.DMA((2,)),
            idx_sems=pltpu.SemaphoreType.DMA((B,)),
        ),
    )
    return run(x.astype(jnp.int32), token_table, position_table)
